# baseline (speedup 1.0000x reference)
import functools

import jax
import jax.numpy as jnp
from jax import lax
from jax.experimental import pallas as pl
from jax.experimental.pallas import tpu as pltpu
from jax.experimental.pallas import tpu_sc as plsc

EMBED_D = 768
NUM_CORES = 2
NUM_SUBCORES = 16
NUM_WORKERS = NUM_CORES * NUM_SUBCORES
B_TOTAL = 4 * 2048
B_PER_W = B_TOTAL // NUM_WORKERS
CHUNK = 64
NCHUNK = B_PER_W // CHUNK

_mesh = plsc.VectorSubcoreMesh(core_axis_name="c", subcore_axis_name="s")


@functools.partial(
    pl.kernel,
    mesh=_mesh,
    out_type=jax.ShapeDtypeStruct((B_TOTAL, EMBED_D), jnp.float32),
    scratch_types=[
        pltpu.VMEM((B_PER_W,), jnp.int32),
        pltpu.VMEM((CHUNK, EMBED_D), jnp.float32),
        pltpu.VMEM((CHUNK, EMBED_D), jnp.float32),
        pltpu.SemaphoreType.DMA,
        pltpu.SemaphoreType.DMA,
        pltpu.SemaphoreType.DMA,
        pltpu.SemaphoreType.DMA,
    ],
)
def _sc_gather(ids_hbm, table_hbm, out_hbm, idx_v, rows0, rows1,
               gsem0, gsem1, ssem0, ssem1):
    wid = lax.axis_index("s") * NUM_CORES + lax.axis_index("c")
    base = wid * B_PER_W
    pltpu.sync_copy(ids_hbm.at[pl.ds(base, B_PER_W)], idx_v)

    rows = (rows0, rows1)
    gsem = (gsem0, gsem1)
    ssem = (ssem0, ssem1)

    def gather(c, buf):
        return pltpu.async_copy(
            table_hbm.at[idx_v.at[pl.ds(c * CHUNK, CHUNK)]], rows[buf],
            gsem[buf])

    def scatter(c, buf):
        return pltpu.async_copy(
            rows[buf], out_hbm.at[pl.ds(base + c * CHUNK, CHUNK)], ssem[buf])

    gd = [None] * NCHUNK
    sd = [None] * NCHUNK
    gd[0] = gather(0, 0)
    gd[1] = gather(1, 1)
    for c in range(NCHUNK):
        buf = c % 2
        gd[c].wait()
        sd[c] = scatter(c, buf)
        if c + 2 < NCHUNK:
            sd[c].wait()
            gd[c + 2] = gather(c + 2, buf)
    if NCHUNK >= 2:
        sd[NCHUNK - 2].wait()
    sd[NCHUNK - 1].wait()


def kernel(input_ids, table):
    b, s = input_ids.shape
    ids = input_ids.reshape(-1).astype(jnp.int32)
    out = _sc_gather(ids, table)
    return out.reshape(b, s, EMBED_D)


# speedup vs baseline: 1.0154x; 1.0154x over previous
import functools

import jax
import jax.numpy as jnp
from jax import lax
from jax.experimental import pallas as pl
from jax.experimental.pallas import tpu as pltpu
from jax.experimental.pallas import tpu_sc as plsc

EMBED_D = 768
NUM_CORES = 2
NUM_SUBCORES = 16
NUM_WORKERS = NUM_CORES * NUM_SUBCORES
B_TOTAL = 4 * 2048
B_PER_W = B_TOTAL // NUM_WORKERS
CHUNK = 32
NCHUNK = B_PER_W // CHUNK
NBUF = 4

_mesh = plsc.VectorSubcoreMesh(core_axis_name="c", subcore_axis_name="s")


@functools.partial(
    pl.kernel,
    mesh=_mesh,
    out_type=jax.ShapeDtypeStruct((B_TOTAL, EMBED_D), jnp.float32),
    scratch_types=[
        pltpu.VMEM((B_PER_W,), jnp.int32),
    ]
    + [pltpu.VMEM((CHUNK, EMBED_D), jnp.float32)] * NBUF
    + [pltpu.SemaphoreType.DMA] * (2 * NBUF),
)
def _sc_gather(ids_hbm, table_hbm, out_hbm, idx_v, *bufs_and_sems):
    rows = bufs_and_sems[:NBUF]
    gsem = bufs_and_sems[NBUF:2 * NBUF]
    ssem = bufs_and_sems[2 * NBUF:]
    wid = lax.axis_index("s") * NUM_CORES + lax.axis_index("c")
    base = wid * B_PER_W
    pltpu.sync_copy(ids_hbm.at[pl.ds(base, B_PER_W)], idx_v)

    def gather(c, buf):
        return pltpu.async_copy(
            table_hbm.at[idx_v.at[pl.ds(c * CHUNK, CHUNK)]], rows[buf],
            gsem[buf])

    def scatter(c, buf):
        return pltpu.async_copy(
            rows[buf], out_hbm.at[pl.ds(base + c * CHUNK, CHUNK)], ssem[buf])

    gd = [None] * NCHUNK
    sd = [None] * NCHUNK
    for c in range(NBUF):
        gd[c] = gather(c, c)
    for c in range(NCHUNK):
        buf = c % NBUF
        gd[c].wait()
        sd[c] = scatter(c, buf)
        if c + NBUF < NCHUNK:
            sd[c].wait()
            gd[c + NBUF] = gather(c + NBUF, buf)
    for c in range(NCHUNK - NBUF, NCHUNK):
        sd[c].wait()


def kernel(input_ids, table):
    b, s = input_ids.shape
    ids = input_ids.reshape(-1).astype(jnp.int32)
    out = _sc_gather(ids, table)
    return out.reshape(b, s, EMBED_D)
